# Initial kernel scaffold; baseline (speedup 1.0000x reference)
import jax
import jax.numpy as jnp
from jax.experimental import pallas as pl


def kernel(x, edge_index, W1, W2, W3, W4):
    raise NotImplementedError("write your pallas kernel here")


# speedup vs baseline: 3.9385x; 3.9385x over previous
import functools

import jax
import jax.numpy as jnp
from jax.experimental import pallas as pl
from jax.experimental.pallas import tpu as pltpu

_N = 10000
_P = 10240


def _mm_kernel(a_ref, b_ref, o_ref, acc_ref, *, nk):
    @pl.when(pl.program_id(2) == 0)
    def _init():
        acc_ref[...] = jnp.zeros_like(acc_ref)

    acc_ref[...] += jnp.dot(a_ref[...], b_ref[...],
                            preferred_element_type=jnp.float32)

    @pl.when(pl.program_id(2) == nk - 1)
    def _out():
        o_ref[...] = acc_ref[...]


def _mm(a, b, bm=512, bn=512, bk=512):
    m, k = a.shape
    k2, n = b.shape
    assert k == k2 and m % 8 == 0
    bm = min(bm, m)
    bn = min(bn, n)
    bk = min(bk, k)
    assert m % bm == 0 and n % bn == 0 and k % bk == 0, (a.shape, b.shape)
    nk = k // bk
    return pl.pallas_call(
        functools.partial(_mm_kernel, nk=nk),
        grid=(m // bm, n // bn, nk),
        in_specs=[
            pl.BlockSpec((bm, bk), lambda i, j, kk: (i, kk)),
            pl.BlockSpec((bk, bn), lambda i, j, kk: (kk, j)),
        ],
        out_specs=pl.BlockSpec((bm, bn), lambda i, j, kk: (i, j)),
        scratch_shapes=[pltpu.VMEM((bm, bn), jnp.float32)],
        out_shape=jax.ShapeDtypeStruct((m, n), jnp.float32),
        compiler_params=pltpu.CompilerParams(
            dimension_semantics=("parallel", "parallel", "arbitrary")),
    )(a, b)


def _act_bn(z):
    a = jnp.where(z > 0, z, jnp.expm1(0.2 * z))
    s = jnp.sum(a, axis=0)
    ss = jnp.sum(a * a, axis=0)
    mean = s / _N
    var = ss / _N - mean * mean
    return (a - mean) / jnp.sqrt(var + 1e-5)


def _pad_to(x, rows, cols):
    return jnp.pad(x, ((0, rows - x.shape[0]), (0, cols - x.shape[1])))


def kernel(x, edge_index, W1, W2, W3, W4):
    loop = jnp.arange(_N, dtype=edge_index.dtype)
    src = jnp.concatenate([edge_index[0], loop])
    dst = jnp.concatenate([edge_index[1], loop])
    deg = jnp.zeros((_N,), jnp.float32).at[dst].add(1.0)
    dinv = jax.lax.rsqrt(deg)
    w_edge = dinv[dst] * dinv[src]
    adj = jnp.zeros((_P, _P), jnp.float32).at[dst, src].add(w_edge)

    xp = _pad_to(x, _P, 128)
    w1p = _pad_to(W1, 128, 512)
    w2p = _pad_to(W2, 512, 512)
    w3p = _pad_to(W3, 512, 2048)
    w4p = _pad_to(W4, 2048, 128)

    h1 = _act_bn(_mm(_mm(adj, xp), w1p))
    h2 = _act_bn(_mm(adj, _mm(h1, w2p)))
    h3 = _act_bn(_mm(_mm(adj, h2), w3p))
    h4 = _act_bn(_mm(adj, _mm(h3, w4p)))

    return (h1[:_N, :500], h2[:_N, :500], h3[:_N, :2000], h4[:_N, :10])
